# baseline (speedup 1.0000x reference)
import functools

import jax
import jax.numpy as jnp
from jax import lax
from jax.experimental import pallas as pl
from jax.experimental.pallas import tpu as pltpu
from jax.experimental.pallas import tpu_sc as plsc

_B = 4096
_H = 200
_D = 64
_O = 5

_IDX_CHUNK = 100
_CHUNKS_PER_ROW = _H // _IDX_CHUNK


def _sc_pool(idx2, embed_table):
  info = plsc.get_sparse_core_info()
  nc, ns = info.num_cores, info.num_subcores
  nw = nc * ns
  b_per_w = _B // nw
  idx_rows_per_w = b_per_w * _CHUNKS_PER_ROW

  mesh = plsc.VectorSubcoreMesh(core_axis_name="c", subcore_axis_name="s")

  @functools.partial(
      pl.kernel,
      out_type=jax.ShapeDtypeStruct((_B, _D), jnp.float32),
      mesh=mesh,
      scratch_types=[
          pltpu.VMEM((idx_rows_per_w, _IDX_CHUNK), jnp.int32),
          pltpu.VMEM((_H, _D), jnp.float32),
          pltpu.VMEM((b_per_w, _D), jnp.float32),
          pltpu.SemaphoreType.DMA,
      ],
      compiler_params=pltpu.CompilerParams(use_tc_tiling_on_sc=False),
  )
  def k(idx_hbm, table_hbm, out_hbm, idx_v, rows_v, pooled_v, sem):
    wid = lax.axis_index("s") * nc + lax.axis_index("c")
    base = wid * b_per_w
    pltpu.sync_copy(idx_hbm.at[pl.ds(base * _CHUNKS_PER_ROW, idx_rows_per_w)],
                    idx_v)

    zero = jnp.zeros((16,), jnp.float32)

    @pl.loop(0, b_per_w)
    def _(i):
      for c in range(_CHUNKS_PER_ROW):
        pltpu.make_async_copy(
            table_hbm.at[idx_v.at[i * _CHUNKS_PER_ROW + c]],
            rows_v.at[pl.ds(c * _IDX_CHUNK, _IDX_CHUNK)],
            sem,
        ).start()
      for c in range(_CHUNKS_PER_ROW):
        pltpu.make_async_copy(
            table_hbm.at[idx_v.at[i * _CHUNKS_PER_ROW + c]],
            rows_v.at[pl.ds(c * _IDX_CHUNK, _IDX_CHUNK)],
            sem,
        ).wait()

      @pl.loop(0, _H, init_carry=(zero, zero, zero, zero), unroll=8)
      def acc_loop(r, carry):
        a0, a1, a2, a3 = carry
        a0 = a0 + rows_v[r, pl.ds(0, 16)]
        a1 = a1 + rows_v[r, pl.ds(16, 16)]
        a2 = a2 + rows_v[r, pl.ds(32, 16)]
        a3 = a3 + rows_v[r, pl.ds(48, 16)]
        return a0, a1, a2, a3

      a0, a1, a2, a3 = acc_loop
      pooled_v[i, pl.ds(0, 16)] = a0
      pooled_v[i, pl.ds(16, 16)] = a1
      pooled_v[i, pl.ds(32, 16)] = a2
      pooled_v[i, pl.ds(48, 16)] = a3

    pltpu.sync_copy(pooled_v, out_hbm.at[pl.ds(base, b_per_w)])

  return k(idx2, embed_table)


def _tc_linear(pooled, bias2, W, b2):
  def body(pooled_ref, bias_ref, w_ref, b_ref, out_ref):
    x = pooled_ref[...] + bias_ref[...]
    out_ref[...] = (
        jnp.dot(x, w_ref[...], preferred_element_type=jnp.float32)
        + b_ref[...]
    )

  return pl.pallas_call(
      body,
      out_shape=jax.ShapeDtypeStruct((_B, _O), jnp.float32),
  )(pooled, bias2, W, b2)


def kernel(inputs, embed_table, bias, W, b):
  idx2 = inputs.astype(jnp.int32).reshape(_B * _CHUNKS_PER_ROW, _IDX_CHUNK)
  pooled = _sc_pool(idx2, embed_table)
  return _tc_linear(pooled, bias.reshape(1, _D), W, b.reshape(1, _O))


# speedup vs baseline: 1.1902x; 1.1902x over previous
import functools

import jax
import jax.numpy as jnp
from jax import lax
from jax.experimental import pallas as pl
from jax.experimental.pallas import tpu as pltpu
from jax.experimental.pallas import tpu_sc as plsc

_B = 4096
_H = 200
_D = 64
_O = 5

_IDX_CHUNK = 100
_CHUNKS_PER_ROW = _H // _IDX_CHUNK

_NBUF = 4


def _sc_pool(idx2, embed_table):
  info = plsc.get_sparse_core_info()
  nc, ns = info.num_cores, info.num_subcores
  nw = nc * ns
  b_per_w = _B // nw
  idx_rows_per_w = b_per_w * _CHUNKS_PER_ROW

  mesh = plsc.VectorSubcoreMesh(core_axis_name="c", subcore_axis_name="s")

  @functools.partial(
      pl.kernel,
      out_type=jax.ShapeDtypeStruct((_B, _D), jnp.float32),
      mesh=mesh,
      scratch_types=[
          pltpu.VMEM((idx_rows_per_w, _IDX_CHUNK), jnp.int32),
          pltpu.VMEM((_NBUF, _H, _D), jnp.float32),
          pltpu.VMEM((b_per_w, _D), jnp.float32),
      ] + [pltpu.SemaphoreType.DMA] * _NBUF,
      compiler_params=pltpu.CompilerParams(use_tc_tiling_on_sc=False),
  )
  def k(idx_hbm, table_hbm, out_hbm, idx_v, rows_v, pooled_v, *sems):
    wid = lax.axis_index("s") * nc + lax.axis_index("c")
    base = wid * b_per_w
    pltpu.sync_copy(idx_hbm.at[pl.ds(base * _CHUNKS_PER_ROW, idx_rows_per_w)],
                    idx_v)

    zero = jnp.zeros((16,), jnp.float32)

    def gather_descs(row, nb):
      return [
          pltpu.make_async_copy(
              table_hbm.at[idx_v.at[row * _CHUNKS_PER_ROW + c]],
              rows_v.at[nb, pl.ds(c * _IDX_CHUNK, _IDX_CHUNK)],
              sems[nb],
          )
          for c in range(_CHUNKS_PER_ROW)
      ]

    def pool_row(row, nb):
      @pl.loop(0, _H, init_carry=(zero, zero, zero, zero), unroll=8)
      def acc_loop(r, carry):
        a0, a1, a2, a3 = carry
        a0 = a0 + rows_v[nb, r, pl.ds(0, 16)]
        a1 = a1 + rows_v[nb, r, pl.ds(16, 16)]
        a2 = a2 + rows_v[nb, r, pl.ds(32, 16)]
        a3 = a3 + rows_v[nb, r, pl.ds(48, 16)]
        return a0, a1, a2, a3

      a0, a1, a2, a3 = acc_loop
      pooled_v[row, pl.ds(0, 16)] = a0
      pooled_v[row, pl.ds(16, 16)] = a1
      pooled_v[row, pl.ds(32, 16)] = a2
      pooled_v[row, pl.ds(48, 16)] = a3

    for nb in range(_NBUF):
      for d in gather_descs(nb, nb):
        d.start()

    @pl.loop(0, b_per_w, step=_NBUF)
    def _(i):
      for nb in range(_NBUF):
        row = i + nb
        for d in gather_descs(row, nb):
          d.wait()
        pool_row(row, nb)

        @pl.when(row + _NBUF < b_per_w)
        def _():
          for d in gather_descs(row + _NBUF, nb):
            d.start()

    pltpu.sync_copy(pooled_v, out_hbm.at[pl.ds(base, b_per_w)])

  return k(idx2, embed_table)


def _tc_linear(pooled, bias2, W, b2):
  def body(pooled_ref, bias_ref, w_ref, b_ref, out_ref):
    x = pooled_ref[...] + bias_ref[...]
    out_ref[...] = (
        jnp.dot(x, w_ref[...], preferred_element_type=jnp.float32)
        + b_ref[...]
    )

  return pl.pallas_call(
      body,
      out_shape=jax.ShapeDtypeStruct((_B, _O), jnp.float32),
  )(pooled, bias2, W, b2)


def kernel(inputs, embed_table, bias, W, b):
  idx2 = inputs.astype(jnp.int32).reshape(_B * _CHUNKS_PER_ROW, _IDX_CHUNK)
  pooled = _sc_pool(idx2, embed_table)
  return _tc_linear(pooled, bias.reshape(1, _D), W, b.reshape(1, _O))


# speedup vs baseline: 1.1962x; 1.0051x over previous
import functools

import jax
import jax.numpy as jnp
from jax import lax
from jax.experimental import pallas as pl
from jax.experimental.pallas import tpu as pltpu
from jax.experimental.pallas import tpu_sc as plsc

_B = 4096
_H = 200
_D = 64
_O = 5

_CHUNK_BOUNDS = (0, 104, 200)
_CHUNKS_PER_ROW = len(_CHUNK_BOUNDS) - 1

_NBUF = 4


def _sc_pool(idx2, embed_table):
  info = plsc.get_sparse_core_info()
  nc, ns = info.num_cores, info.num_subcores
  nw = nc * ns
  b_per_w = _B // nw

  mesh = plsc.VectorSubcoreMesh(core_axis_name="c", subcore_axis_name="s")

  @functools.partial(
      pl.kernel,
      out_type=jax.ShapeDtypeStruct((_B, _D), jnp.float32),
      mesh=mesh,
      scratch_types=[
          pltpu.VMEM((b_per_w, _H), jnp.int32),
          pltpu.VMEM((_NBUF, _H, _D), jnp.float32),
          pltpu.VMEM((b_per_w, _D), jnp.float32),
      ] + [pltpu.SemaphoreType.DMA] * _NBUF,
      compiler_params=pltpu.CompilerParams(use_tc_tiling_on_sc=False),
  )
  def k(idx_hbm, table_hbm, out_hbm, idx_v, rows_v, pooled_v, *sems):
    wid = lax.axis_index("s") * nc + lax.axis_index("c")
    base = wid * b_per_w
    pltpu.sync_copy(idx_hbm.at[pl.ds(base, b_per_w)], idx_v)

    zero = jnp.zeros((16,), jnp.float32)

    def gather_descs(row, nb):
      return [
          pltpu.make_async_copy(
              table_hbm.at[idx_v.at[row, pl.ds(lo, hi - lo)]],
              rows_v.at[nb, pl.ds(lo, hi - lo)],
              sems[nb],
          )
          for lo, hi in zip(_CHUNK_BOUNDS[:-1], _CHUNK_BOUNDS[1:])
      ]

    def pool_row(row, nb):
      @pl.loop(0, _H, init_carry=(zero, zero, zero, zero), unroll=8)
      def acc_loop(r, carry):
        a0, a1, a2, a3 = carry
        a0 = a0 + rows_v[nb, r, pl.ds(0, 16)]
        a1 = a1 + rows_v[nb, r, pl.ds(16, 16)]
        a2 = a2 + rows_v[nb, r, pl.ds(32, 16)]
        a3 = a3 + rows_v[nb, r, pl.ds(48, 16)]
        return a0, a1, a2, a3

      a0, a1, a2, a3 = acc_loop
      pooled_v[row, pl.ds(0, 16)] = a0
      pooled_v[row, pl.ds(16, 16)] = a1
      pooled_v[row, pl.ds(32, 16)] = a2
      pooled_v[row, pl.ds(48, 16)] = a3

    for nb in range(_NBUF):
      for d in gather_descs(nb, nb):
        d.start()

    @pl.loop(0, b_per_w, step=_NBUF)
    def _(i):
      for nb in range(_NBUF):
        row = i + nb
        for d in gather_descs(row, nb):
          d.wait()
        pool_row(row, nb)

        @pl.when(row + _NBUF < b_per_w)
        def _():
          for d in gather_descs(row + _NBUF, nb):
            d.start()

    pltpu.sync_copy(pooled_v, out_hbm.at[pl.ds(base, b_per_w)])

  return k(idx2, embed_table)


def _tc_linear(pooled, bias2, W, b2):
  def body(pooled_ref, bias_ref, w_ref, b_ref, out_ref):
    x = pooled_ref[...] + bias_ref[...]
    out_ref[...] = (
        jnp.dot(x, w_ref[...], preferred_element_type=jnp.float32)
        + b_ref[...]
    )

  return pl.pallas_call(
      body,
      out_shape=jax.ShapeDtypeStruct((_B, _O), jnp.float32),
  )(pooled, bias2, W, b2)


def kernel(inputs, embed_table, bias, W, b):
  pooled = _sc_pool(inputs.astype(jnp.int32), embed_table)
  return _tc_linear(pooled, bias.reshape(1, _D), W, b.reshape(1, _O))
